# Initial kernel scaffold; baseline (speedup 1.0000x reference)
import jax
import jax.numpy as jnp
from jax.experimental import pallas as pl


def kernel(diag_ids, drug_ids, age_ids, race_ids, gender_ids, icd_emb, drug_emb, age_emb, race_emb, gender_emb, W1, b1, g1, beta1, W2, b2, g2, beta2, W3, b3):
    raise NotImplementedError("write your pallas kernel here")


# speedup vs baseline: 5.4893x; 5.4893x over previous
import functools

import jax
import jax.numpy as jnp
from jax import lax
from jax.experimental import pallas as pl
from jax.experimental.pallas import tpu as pltpu
from jax.experimental.pallas import tpu_sc as plsc

B = 4096
EMB = 64
L_DIAG = 50
L_DRUG = 20
N_FIELDS = 3 + L_DIAG + L_DRUG
IN_DIM = N_FIELDS * EMB
H1, H2 = 512, 256

_NW = 32
_CHUNK = 128
_ROWS = B * N_FIELDS
_CHUNKS = _ROWS // _CHUNK
_CPW = _CHUNKS // _NW
_BB = 512
_NB = B // _BB


def _sc_gather(table, ids_flat):
    mesh = plsc.VectorSubcoreMesh(core_axis_name="c", subcore_axis_name="s")
    ipw = _CPW * _CHUNK

    @functools.partial(
        pl.kernel,
        mesh=mesh,
        compiler_params=pltpu.CompilerParams(use_tc_tiling_on_sc=False),
        out_type=jax.ShapeDtypeStruct((_ROWS, EMB), jnp.float32),
        scratch_types=[
            pltpu.VMEM((_CPW * _CHUNK,), jnp.int32),
            pltpu.VMEM((_CHUNK, EMB), jnp.float32),
            pltpu.VMEM((_CHUNK, EMB), jnp.float32),
            pltpu.SemaphoreType.DMA,
            pltpu.SemaphoreType.DMA,
        ],
    )
    def gather_k(table_hbm, ids_hbm, out_hbm, idx_v, buf0, buf1, sem0, sem1):
        wid = lax.axis_index("s") * 2 + lax.axis_index("c")
        c0 = wid * _CPW
        pltpu.sync_copy(ids_hbm.at[pl.ds(wid * ipw, ipw)], idx_v)

        def pair(i, _):
            j0 = 2 * i
            j1 = 2 * i + 1
            ca = pltpu.async_copy(
                table_hbm.at[idx_v.at[pl.ds(j0 * _CHUNK, _CHUNK)]], buf0, sem0)
            cb = pltpu.async_copy(
                table_hbm.at[idx_v.at[pl.ds(j1 * _CHUNK, _CHUNK)]], buf1, sem1)
            ca.wait()
            pltpu.sync_copy(buf0, out_hbm.at[pl.ds((c0 + j0) * _CHUNK, _CHUNK)])
            cb.wait()
            pltpu.sync_copy(buf1, out_hbm.at[pl.ds((c0 + j1) * _CHUNK, _CHUNK)])
            return 0

        lax.fori_loop(0, _CPW // 2, pair, 0)
        jt = _CPW - 1
        pltpu.async_copy(
            table_hbm.at[idx_v.at[pl.ds(jt * _CHUNK, _CHUNK)]], buf0, sem0).wait()
        pltpu.sync_copy(buf0, out_hbm.at[pl.ds((c0 + jt) * _CHUNK, _CHUNK)])

    return gather_k(table, ids_flat)


def _mlp_body(x_ref, w1_ref, b1_ref, g1_ref, bt1_ref, w2_ref, b2_ref,
              g2_ref, bt2_ref, w3_ref, b3_ref, out_ref,
              h1_ref, s1_ref, s2_ref):
    i = pl.program_id(0)
    h = jnp.dot(x_ref[...], w1_ref[...], preferred_element_type=jnp.float32)
    h = jnp.maximum(h + b1_ref[...], 0.0)
    h1_ref[pl.ds(i * _BB, _BB), :] = h
    colsum = jnp.sum(h, axis=0, keepdims=True)
    colsq = jnp.sum(h * h, axis=0, keepdims=True)

    @pl.when(i == 0)
    def _():
        s1_ref[...] = colsum
        s2_ref[...] = colsq

    @pl.when(i > 0)
    def _():
        s1_ref[...] = s1_ref[...] + colsum
        s2_ref[...] = s2_ref[...] + colsq

    @pl.when(i == _NB - 1)
    def _():
        inv_b = 1.0 / B
        mean = s1_ref[...] * inv_b
        var = s2_ref[...] * inv_b - mean * mean
        scale = g1_ref[...] * lax.rsqrt(var + 1e-5)
        shift = bt1_ref[...] - mean * scale
        h1n = h1_ref[...] * scale + shift
        h2 = jnp.dot(h1n, w2_ref[...], preferred_element_type=jnp.float32)
        h2 = jnp.maximum(h2 + b2_ref[...], 0.0)
        m2 = jnp.sum(h2, axis=0, keepdims=True) * inv_b
        v2 = jnp.sum(h2 * h2, axis=0, keepdims=True) * inv_b - m2 * m2
        sc2 = g2_ref[...] * lax.rsqrt(v2 + 1e-5)
        sh2 = bt2_ref[...] - m2 * sc2
        h2n = h2 * sc2 + sh2
        out_ref[...] = (jnp.sum(h2n * w3_ref[...], axis=1, keepdims=True)
                        + b3_ref[...])


def _mlp(x, W1, b1, g1, beta1, W2, b2, g2, beta2, W3, b3):
    full = lambda shape: pl.BlockSpec(shape, lambda i: (0, 0))
    return pl.pallas_call(
        _mlp_body,
        grid=(_NB,),
        in_specs=[
            pl.BlockSpec((_BB, IN_DIM), lambda i: (i, 0)),
            full((IN_DIM, H1)),
            full((1, H1)), full((1, H1)), full((1, H1)),
            full((H1, H2)),
            full((1, H2)), full((1, H2)), full((1, H2)),
            full((1, H2)),
            full((1, 1)),
        ],
        out_specs=full((B, 1)),
        out_shape=jax.ShapeDtypeStruct((B, 1), jnp.float32),
        scratch_shapes=[
            pltpu.VMEM((B, H1), jnp.float32),
            pltpu.VMEM((1, H1), jnp.float32),
            pltpu.VMEM((1, H1), jnp.float32),
        ],
    )(x, W1, b1.reshape(1, H1), g1.reshape(1, H1), beta1.reshape(1, H1),
      W2, b2.reshape(1, H2), g2.reshape(1, H2), beta2.reshape(1, H2),
      W3.reshape(1, H2), b3.reshape(1, 1))


def kernel(diag_ids, drug_ids, age_ids, race_ids, gender_ids,
           icd_emb, drug_emb, age_emb, race_emb, gender_emb,
           W1, b1, g1, beta1, W2, b2, g2, beta2, W3, b3):
    v_icd = icd_emb.shape[0]
    v_drug = drug_emb.shape[0]
    v_age = age_emb.shape[0]
    v_race = race_emb.shape[0]
    table = jnp.concatenate(
        [icd_emb, drug_emb, age_emb, race_emb, gender_emb], axis=0)
    o_drug = v_icd
    o_age = o_drug + v_drug
    o_race = o_age + v_age
    o_gender = o_race + v_race
    ids = jnp.concatenate(
        [age_ids[:, None] + o_age,
         race_ids[:, None] + o_race,
         gender_ids[:, None] + o_gender,
         diag_ids,
         drug_ids + o_drug], axis=1).astype(jnp.int32)
    x = _sc_gather(table, ids.reshape(_ROWS)).reshape(B, IN_DIM)
    out = _mlp(x, W1, b1, g1, beta1, W2, b2, g2, beta2, W3, b3)
    return out[:, 0]
